# Initial kernel scaffold; baseline (speedup 1.0000x reference)
import jax
import jax.numpy as jnp
from jax.experimental import pallas as pl


def kernel(x, sadj, fadj, madj, S_W1, S_b1, S_W2, S_b2, F_W1, F_b1, F_W2, F_b2, M_W1, M_b1, M_W2, M_b2, C_W1, C_b1, C_W2, C_b2, att_W1, att_b1, att_W2, mlp_W, mlp_b, dec_W1, dec_b1, dec_Wd, dec_bd, dec_Wm, dec_bm):
    raise NotImplementedError("write your pallas kernel here")


# speedup vs baseline: 1.7974x; 1.7974x over previous
import functools

import jax
import jax.numpy as jnp
from jax.experimental import pallas as pl
from jax.experimental.pallas import tpu as pltpu

_BM = 400


def _prep_body(x_ref, w_ref, o_ref):
    o_ref[...] = jnp.dot(x_ref[...], w_ref[...],
                         preferred_element_type=jnp.float32)


def _stage1_body(adj_ref, u_ref, b1_ref, w2_ref, v_ref):
    h = jnp.dot(adj_ref[...], u_ref[...], preferred_element_type=jnp.float32)
    h = jnp.maximum(h + b1_ref[...], 0.0)
    v_ref[...] = jnp.dot(h, w2_ref[...], preferred_element_type=jnp.float32)


def _stage2_body(adj_ref, v_ref, b2_ref, o_ref):
    o_ref[...] = (jnp.dot(adj_ref[...], v_ref[...],
                          preferred_element_type=jnp.float32) + b2_ref[...])


def _tail_body(os_ref, of_ref, om_ref, aw1_ref, ab1_ref, aw2_ref,
               mw_ref, mb_ref, dw1_ref, db1_ref, dwd_ref, dbd_ref,
               dwm_ref, dbm_ref, emb_ref, disp_ref, mean_ref):
    o_s = os_ref[...]
    o_f = of_ref[...]
    o_m = om_ref[...]
    emb1 = o_s[:, :32]
    emb2 = o_f[:, :32]
    emb3 = o_m[:, :32]
    xcom = (o_s[:, 32:] + o_f[:, 32:] + o_m[:, 32:]) * (1.0 / 3.0)

    aw1 = aw1_ref[...]
    ab1 = ab1_ref[...]
    aw2 = aw2_ref[...]

    def att(z):
        t = jnp.tanh(jnp.dot(z, aw1, preferred_element_type=jnp.float32)
                     + ab1)
        return jnp.dot(t, aw2, preferred_element_type=jnp.float32)

    w1 = att(emb1)
    w2 = att(emb2)
    w3 = att(emb3)
    w4 = att(xcom)
    m = jnp.maximum(jnp.maximum(w1, w2), jnp.maximum(w3, w4))
    e1 = jnp.exp(w1 - m)
    e2 = jnp.exp(w2 - m)
    e3 = jnp.exp(w3 - m)
    e4 = jnp.exp(w4 - m)
    denom = e1 + e2 + e3 + e4
    emb = (e1 * emb1 + e2 * emb2 + e3 * emb3 + e4 * xcom) / denom

    emb = jnp.dot(emb, mw_ref[...], preferred_element_type=jnp.float32) \
        + mb_ref[...]
    emb_ref[...] = emb

    h = jnp.maximum(
        jnp.dot(emb, dw1_ref[...], preferred_element_type=jnp.float32)
        + db1_ref[...], 0.0)
    sd = jnp.dot(h, dwd_ref[...], preferred_element_type=jnp.float32) \
        + dbd_ref[...]
    disp_ref[...] = jnp.maximum(sd, 0.0) + jnp.log1p(jnp.exp(-jnp.abs(sd)))
    sm = jnp.dot(h, dwm_ref[...], preferred_element_type=jnp.float32) \
        + dbm_ref[...]
    mean_ref[...] = jnp.exp(jnp.clip(sm, -15.0, 15.0))


def _full(shape):
    return pl.BlockSpec(shape, lambda i: (0,) * len(shape))


def _rows(shape):
    ndim = len(shape)
    return pl.BlockSpec(shape, lambda i: (i,) + (0,) * (ndim - 1))


def _gcn_pair(adj, u, b1cat, w2bd, b2cat, n):
    grid = (n // _BM,)
    v = pl.pallas_call(
        _stage1_body,
        grid=grid,
        in_specs=[_rows((_BM, n)), _full((n, 128)), _full((1, 128)),
                  _full((128, 64))],
        out_specs=_rows((_BM, 64)),
        out_shape=jax.ShapeDtypeStruct((n, 64), jnp.float32),
        compiler_params=pltpu.CompilerParams(
            dimension_semantics=("arbitrary",)),
    )(adj, u, b1cat, w2bd)
    o = pl.pallas_call(
        _stage2_body,
        grid=grid,
        in_specs=[_rows((_BM, n)), _full((n, 64)), _full((1, 64))],
        out_specs=_rows((_BM, 64)),
        out_shape=jax.ShapeDtypeStruct((n, 64), jnp.float32),
        compiler_params=pltpu.CompilerParams(
            dimension_semantics=("arbitrary",)),
    )(adj, v, b2cat)
    return o


def kernel(x, sadj, fadj, madj, S_W1, S_b1, S_W2, S_b2, F_W1, F_b1, F_W2,
           F_b2, M_W1, M_b1, M_W2, M_b2, C_W1, C_b1, C_W2, C_b2, att_W1,
           att_b1, att_W2, mlp_W, mlp_b, dec_W1, dec_b1, dec_Wd, dec_bd,
           dec_Wm, dec_bm):
    n, nfeat = x.shape

    w1cat = jnp.concatenate([S_W1, F_W1, M_W1, C_W1], axis=1)
    p = pl.pallas_call(
        _prep_body,
        grid=(1,),
        in_specs=[_full((n, nfeat)), _full((nfeat, 256))],
        out_specs=_full((n, 256)),
        out_shape=jax.ShapeDtypeStruct((n, 256), jnp.float32),
    )(x, w1cat)

    zeros64 = jnp.zeros((64, 32), jnp.float32)
    outs = []
    for gi, (adj, w2, b1, b2) in enumerate([
            (sadj, S_W2, S_b1, S_b2),
            (fadj, F_W2, F_b1, F_b2),
            (madj, M_W2, M_b1, M_b2)]):
        u = jnp.concatenate([p[:, gi * 64:(gi + 1) * 64], p[:, 192:256]],
                            axis=1)
        b1cat = jnp.concatenate([b1, C_b1]).reshape(1, 128)
        b2cat = jnp.concatenate([b2, C_b2]).reshape(1, 64)
        w2bd = jnp.concatenate([
            jnp.concatenate([w2, zeros64], axis=1),
            jnp.concatenate([zeros64, C_W2], axis=1)], axis=0)
        outs.append(_gcn_pair(adj, u, b1cat, w2bd, b2cat, n))
    o_s, o_f, o_m = outs

    bt = 2000
    emb, disp, mean = pl.pallas_call(
        _tail_body,
        grid=(n // bt,),
        in_specs=[_rows((bt, 64)), _rows((bt, 64)), _rows((bt, 64)),
                  _full((32, 16)), _full((1, 16)), _full((16, 1)),
                  _full((32, 32)), _full((1, 32)),
                  _full((32, 64)), _full((1, 64)),
                  _full((64, 128)), _full((1, 128)),
                  _full((64, 128)), _full((1, 128))],
        out_specs=[_rows((bt, 32)), _rows((bt, 128)), _rows((bt, 128))],
        out_shape=[jax.ShapeDtypeStruct((n, 32), jnp.float32),
                   jax.ShapeDtypeStruct((n, 128), jnp.float32),
                   jax.ShapeDtypeStruct((n, 128), jnp.float32)],
        compiler_params=pltpu.CompilerParams(
            dimension_semantics=("arbitrary",)),
    )(o_s, o_f, o_m, att_W1, att_b1.reshape(1, 16), att_W2,
      mlp_W, mlp_b.reshape(1, 32), dec_W1, dec_b1.reshape(1, 64),
      dec_Wd, dec_bd.reshape(1, 128), dec_Wm, dec_bm.reshape(1, 128))

    com1 = o_s[:, 32:]
    com2 = o_f[:, 32:]
    com3 = o_m[:, 32:]
    return (com1, com2, com3, emb, disp, mean)


# speedup vs baseline: 1.8875x; 1.0501x over previous
import functools

import jax
import jax.numpy as jnp
from jax.experimental import pallas as pl
from jax.experimental.pallas import tpu as pltpu

_BM = 400


def _prep_body(x_ref, w_ref, us_ref, uf_ref, um_ref):
    p = jnp.dot(x_ref[...], w_ref[...], preferred_element_type=jnp.float32)
    xc = p[:, 192:256]
    us_ref[...] = jnp.concatenate([p[:, 0:64], xc], axis=1)
    uf_ref[...] = jnp.concatenate([p[:, 64:128], xc], axis=1)
    um_ref[...] = jnp.concatenate([p[:, 128:192], xc], axis=1)


def _gcn_body(adj_ref, u_ref, b1_ref, w2_ref, b2_ref, o_ref, v_ref):
    phase = pl.program_id(0)
    i = pl.program_id(1)

    @pl.when(phase == 0)
    def _layer1():
        h = jnp.dot(adj_ref[...], u_ref[...],
                    preferred_element_type=jnp.float32)
        h = jnp.maximum(h + b1_ref[...], 0.0)
        v_ref[pl.ds(i * _BM, _BM), :] = jnp.dot(
            h, w2_ref[...], preferred_element_type=jnp.float32)

    @pl.when(phase == 1)
    def _layer2():
        o_ref[...] = (jnp.dot(adj_ref[...], v_ref[...],
                              preferred_element_type=jnp.float32)
                      + b2_ref[...])


def _tail_body(os_ref, of_ref, om_ref, aw1_ref, ab1_ref, aw2_ref,
               mw_ref, mb_ref, dw1_ref, db1_ref, dwd_ref, dbd_ref,
               dwm_ref, dbm_ref, emb_ref, disp_ref, mean_ref):
    o_s = os_ref[...]
    o_f = of_ref[...]
    o_m = om_ref[...]
    emb1 = o_s[:, :32]
    emb2 = o_f[:, :32]
    emb3 = o_m[:, :32]
    xcom = (o_s[:, 32:] + o_f[:, 32:] + o_m[:, 32:]) * (1.0 / 3.0)

    aw1 = aw1_ref[...]
    ab1 = ab1_ref[...]
    aw2 = aw2_ref[...]

    def att(z):
        t = jnp.tanh(jnp.dot(z, aw1, preferred_element_type=jnp.float32)
                     + ab1)
        return jnp.dot(t, aw2, preferred_element_type=jnp.float32)

    w1 = att(emb1)
    w2 = att(emb2)
    w3 = att(emb3)
    w4 = att(xcom)
    m = jnp.maximum(jnp.maximum(w1, w2), jnp.maximum(w3, w4))
    e1 = jnp.exp(w1 - m)
    e2 = jnp.exp(w2 - m)
    e3 = jnp.exp(w3 - m)
    e4 = jnp.exp(w4 - m)
    denom = e1 + e2 + e3 + e4
    emb = (e1 * emb1 + e2 * emb2 + e3 * emb3 + e4 * xcom) / denom

    emb = jnp.dot(emb, mw_ref[...], preferred_element_type=jnp.float32) \
        + mb_ref[...]
    emb_ref[...] = emb

    h = jnp.maximum(
        jnp.dot(emb, dw1_ref[...], preferred_element_type=jnp.float32)
        + db1_ref[...], 0.0)
    sd = jnp.dot(h, dwd_ref[...], preferred_element_type=jnp.float32) \
        + dbd_ref[...]
    disp_ref[...] = jnp.maximum(sd, 0.0) + jnp.log1p(jnp.exp(-jnp.abs(sd)))
    sm = jnp.dot(h, dwm_ref[...], preferred_element_type=jnp.float32) \
        + dbm_ref[...]
    mean_ref[...] = jnp.exp(jnp.clip(sm, -15.0, 15.0))


def _full(shape, ng=1):
    return pl.BlockSpec(shape, lambda *g: (0,) * len(shape))


def _rows(shape):
    ndim = len(shape)
    return pl.BlockSpec(shape, lambda i: (i,) + (0,) * (ndim - 1))


def _gcn_pair(adj, u, b1cat, w2bd, b2cat, n):
    grid = (2, n // _BM)
    return pl.pallas_call(
        _gcn_body,
        grid=grid,
        in_specs=[pl.BlockSpec((_BM, n), lambda p, i: (i, 0)),
                  _full((n, 128)), _full((1, 128)),
                  _full((128, 64)), _full((1, 64))],
        out_specs=pl.BlockSpec((_BM, 64), lambda p, i: (i, 0)),
        out_shape=jax.ShapeDtypeStruct((n, 64), jnp.float32),
        scratch_shapes=[pltpu.VMEM((n, 64), jnp.float32)],
        compiler_params=pltpu.CompilerParams(
            dimension_semantics=("arbitrary", "arbitrary")),
    )(adj, u, b1cat, w2bd, b2cat)


def kernel(x, sadj, fadj, madj, S_W1, S_b1, S_W2, S_b2, F_W1, F_b1, F_W2,
           F_b2, M_W1, M_b1, M_W2, M_b2, C_W1, C_b1, C_W2, C_b2, att_W1,
           att_b1, att_W2, mlp_W, mlp_b, dec_W1, dec_b1, dec_Wd, dec_bd,
           dec_Wm, dec_bm):
    n, nfeat = x.shape

    w1cat = jnp.concatenate([S_W1, F_W1, M_W1, C_W1], axis=1)
    bp = 1000
    u_s, u_f, u_m = pl.pallas_call(
        _prep_body,
        grid=(n // bp,),
        in_specs=[_rows((bp, nfeat)), _full((nfeat, 256))],
        out_specs=[_rows((bp, 128))] * 3,
        out_shape=[jax.ShapeDtypeStruct((n, 128), jnp.float32)] * 3,
    )(x, w1cat)

    zeros64 = jnp.zeros((64, 32), jnp.float32)
    outs = []
    for adj, u, w2, b1, b2 in [
            (sadj, u_s, S_W2, S_b1, S_b2),
            (fadj, u_f, F_W2, F_b1, F_b2),
            (madj, u_m, M_W2, M_b1, M_b2)]:
        b1cat = jnp.concatenate([b1, C_b1]).reshape(1, 128)
        b2cat = jnp.concatenate([b2, C_b2]).reshape(1, 64)
        w2bd = jnp.concatenate([
            jnp.concatenate([w2, zeros64], axis=1),
            jnp.concatenate([zeros64, C_W2], axis=1)], axis=0)
        outs.append(_gcn_pair(adj, u, b1cat, w2bd, b2cat, n))
    o_s, o_f, o_m = outs

    bt = 2000
    emb, disp, mean = pl.pallas_call(
        _tail_body,
        grid=(n // bt,),
        in_specs=[_rows((bt, 64)), _rows((bt, 64)), _rows((bt, 64)),
                  _full((32, 16)), _full((1, 16)), _full((16, 1)),
                  _full((32, 32)), _full((1, 32)),
                  _full((32, 64)), _full((1, 64)),
                  _full((64, 128)), _full((1, 128)),
                  _full((64, 128)), _full((1, 128))],
        out_specs=[_rows((bt, 32)), _rows((bt, 128)), _rows((bt, 128))],
        out_shape=[jax.ShapeDtypeStruct((n, 32), jnp.float32),
                   jax.ShapeDtypeStruct((n, 128), jnp.float32),
                   jax.ShapeDtypeStruct((n, 128), jnp.float32)],
        compiler_params=pltpu.CompilerParams(
            dimension_semantics=("arbitrary",)),
    )(o_s, o_f, o_m, att_W1, att_b1.reshape(1, 16), att_W2,
      mlp_W, mlp_b.reshape(1, 32), dec_W1, dec_b1.reshape(1, 64),
      dec_Wd, dec_bd.reshape(1, 128), dec_Wm, dec_bm.reshape(1, 128))

    com1 = o_s[:, 32:]
    com2 = o_f[:, 32:]
    com3 = o_m[:, 32:]
    return (com1, com2, com3, emb, disp, mean)
